# baseline (speedup 1.0000x reference)
import jax
import jax.numpy as jnp
from jax import lax
from jax.experimental import pallas as pl
from jax.experimental.pallas import tpu as pltpu
from jax.experimental.pallas import tpu_sc as plsc

VOCAB = 100000
DIM = 64
BATCH = 4096
HIST = 50
FEAT = 16

NC = 2
NS = 16
NW = NC * NS
BPW = BATCH // NW
NG = BPW
NBUF = 8
NT = NG // NBUF

_MLP_BLK = 512


def _sc_body(tab_hbm, x_hbm, out_hbm, idx_v, rows_v, out_v, *sems):
    wid = lax.axis_index("s") * NC + lax.axis_index("c")

    pltpu.sync_copy(x_hbm.at[pl.ds(wid * BPW, BPW)], idx_v)
    inv_h = jnp.float32(1.0 / HIST)

    for b in range(NBUF):
        pltpu.async_copy(tab_hbm.at[idx_v.at[b]], rows_v.at[b], sems[b])

    def outer(t, carry):
        for b in range(NBUF):
            g = t * NBUF + b
            pltpu.make_async_copy(tab_hbm.at[idx_v.at[g]], rows_v.at[b],
                                  sems[b]).wait()
            for c in range(DIM // 16):
                lo, hi = c * 16, (c + 1) * 16
                a0 = rows_v[b, 0, lo:hi]
                a1 = rows_v[b, 1, lo:hi]
                for j in range(2, HIST, 2):
                    a0 = a0 + rows_v[b, j, lo:hi]
                    a1 = a1 + rows_v[b, j + 1, lo:hi]
                out_v[g, lo:hi] = (a0 + a1) * inv_h
            @pl.when(g + NBUF < NG)
            def _():
                pltpu.async_copy(tab_hbm.at[idx_v.at[g + NBUF]], rows_v.at[b],
                                 sems[b])
        return carry

    lax.fori_loop(0, NT, outer, 0)
    pltpu.sync_copy(out_v, out_hbm.at[pl.ds(wid * BPW, BPW)])


def _sc_pool(table, x_grouped):
    mesh = plsc.VectorSubcoreMesh(core_axis_name="c", subcore_axis_name="s")
    kfn = pl.kernel(
        _sc_body,
        out_type=jax.ShapeDtypeStruct((BATCH, DIM), jnp.float32),
        mesh=mesh,
        scratch_types=[
            pltpu.VMEM((NG, HIST), jnp.int32),
            pltpu.VMEM((NBUF, HIST, DIM), jnp.float32),
            pltpu.VMEM((BPW, DIM), jnp.float32),
        ] + [pltpu.SemaphoreType.DMA] * NBUF,
        compiler_params=pltpu.CompilerParams(use_tc_tiling_on_sc=False),
    )
    return kfn(table, x_grouped)


def _mlp(mean, W1, b1, W2, b2):
    def body(m_ref, w1_ref, b1_ref, w2_ref, b2_ref, o_ref):
        h = jnp.dot(m_ref[...], w1_ref[...],
                    preferred_element_type=jnp.float32) + b1_ref[...]
        h = jnp.maximum(h, 0.0)
        o_ref[...] = jnp.dot(h, w2_ref[...],
                             preferred_element_type=jnp.float32) + b2_ref[...]

    return pl.pallas_call(
        body,
        grid=(BATCH // _MLP_BLK,),
        in_specs=[
            pl.BlockSpec((_MLP_BLK, DIM), lambda i: (i, 0)),
            pl.BlockSpec((DIM, FEAT), lambda i: (0, 0)),
            pl.BlockSpec((1, FEAT), lambda i: (0, 0)),
            pl.BlockSpec((FEAT, 1), lambda i: (0, 0)),
            pl.BlockSpec((1, 1), lambda i: (0, 0)),
        ],
        out_specs=pl.BlockSpec((_MLP_BLK, 1), lambda i: (i, 0)),
        out_shape=jax.ShapeDtypeStruct((BATCH, 1), jnp.float32),
    )(mean, W1, b1.reshape(1, FEAT), W2, b2.reshape(1, 1))


def kernel(x, table, W1, b1, W2, b2):
    mean = _sc_pool(table, x.astype(jnp.int32))
    return _mlp(mean, W1, b1, W2, b2)
